# baseline (speedup 1.0000x reference)
import functools

import jax
import jax.numpy as jnp
from jax import lax
from jax.experimental import pallas as pl
from jax.experimental.pallas import tpu as pltpu
from jax.experimental.pallas import tpu_sc as plsc

NC, NS = 2, 16
NW = NC * NS
CH = 96

_MESH = plsc.VectorSubcoreMesh(
    core_axis_name="c", subcore_axis_name="s", num_cores=NC, num_subcores=NS
)


def _deg_kernel_fn(npad, nch, rpt, feat):
    @functools.partial(
        pl.kernel,
        out_type=jax.ShapeDtypeStruct((NC, npad, feat), jnp.float32),
        mesh=_MESH,
        scratch_types=[
            pltpu.VMEM((nch, CH), jnp.int32),
            pltpu.VMEM((CH, feat), jnp.float32),
            pltpu.VMEM_SHARED((npad, feat), jnp.float32),
        ],
    )
    def deg_kernel(dst_hbm, ones_hbm, zeros_hbm, deg_out, dst_v, ones_v, deg_sh):
        c = lax.axis_index("c")
        s = lax.axis_index("s")
        gt = c * NS + s
        pltpu.sync_copy(dst_hbm.at[gt], dst_v)
        pltpu.sync_copy(ones_hbm, ones_v)
        base = pl.multiple_of(s * rpt, 8)
        pltpu.sync_copy(zeros_hbm, deg_sh.at[pl.ds(base, rpt)])
        plsc.subcore_barrier()

        def body(j, carry):
            pltpu.sync_copy(ones_v, deg_sh.at[dst_v.at[j]], add=True)
            return carry

        lax.fori_loop(0, nch, body, 0)
        plsc.subcore_barrier()
        pltpu.sync_copy(deg_sh.at[pl.ds(base, rpt)],
                        deg_out.at[c, pl.ds(base, rpt)])

    return deg_kernel


def _scatter_kernel_fn(npad, nch, rpt, ept, feat):
    @functools.partial(
        pl.kernel,
        out_type=jax.ShapeDtypeStruct((NC, npad, feat), jnp.float32),
        mesh=_MESH,
        scratch_types=[
            pltpu.VMEM((ept,), jnp.int32),
            pltpu.VMEM((nch, CH), jnp.int32),
            pltpu.VMEM((2, CH, feat), jnp.float32),
            pltpu.VMEM_SHARED((npad, feat), jnp.float32),
            pltpu.SemaphoreType.DMA,
        ],
    )
    def scatter_kernel(hs_hbm, src_hbm, dst_hbm, zeros_hbm, acc_out,
                       src_v, dst_v, rows_v, acc_sh, sem_g):
        c = lax.axis_index("c")
        s = lax.axis_index("s")
        gt = c * NS + s
        pltpu.sync_copy(src_hbm.at[gt], src_v)
        pltpu.sync_copy(dst_hbm.at[gt], dst_v)
        base = pl.multiple_of(s * rpt, 8)
        pltpu.sync_copy(zeros_hbm, acc_sh.at[pl.ds(base, rpt)])
        plsc.subcore_barrier()

        def issue(j, par):
            off = pl.multiple_of(j * CH, 8)
            pltpu.async_copy(hs_hbm.at[src_v.at[pl.ds(off, CH)]],
                             rows_v.at[par], sem_g)

        def wait(j, par):
            off = pl.multiple_of(j * CH, 8)
            pltpu.make_async_copy(
                hs_hbm.at[src_v.at[pl.ds(off, CH)]],
                rows_v.at[par], sem_g).wait()

        issue(0, 0)

        def body(j, carry):
            par = lax.rem(j, 2)
            wait(j, par)
            issue(j + 1, 1 - par)
            pltpu.sync_copy(rows_v.at[par], acc_sh.at[dst_v.at[j]],
                            add=True)
            return carry

        lax.fori_loop(0, nch - 1, body, 0)
        last = nch - 1
        lpar = lax.rem(last, 2)
        wait(last, lpar)
        pltpu.sync_copy(rows_v.at[lpar], acc_sh.at[dst_v.at[last]], add=True)
        plsc.subcore_barrier()
        pltpu.sync_copy(acc_sh.at[pl.ds(base, rpt)],
                        acc_out.at[c, pl.ds(base, rpt)])

    return scatter_kernel


def _mm_body(x_ref, w_ref, dp_ref, hs_ref):
    h = jnp.dot(x_ref[...], w_ref[...], preferred_element_type=jnp.float32)
    dis = lax.rsqrt(dp_ref[0] + dp_ref[1] + 1.0)
    hs_ref[...] = h * dis


def _combine_body(p_ref, dp_ref, hs_ref, b_ref, o_ref):
    dis = lax.rsqrt(dp_ref[0] + dp_ref[1] + 1.0)
    o_ref[...] = dis * (p_ref[0] + p_ref[1] + hs_ref[...]) + b_ref[...]


def kernel(x, edge_index, W, b):
    n, feat = x.shape
    e = edge_index.shape[1]
    npad = ((n + 1 + NW * 16 - 1) // (NW * 16)) * (NW * 16)
    rpt = npad // NS
    ept = ((e + NW * CH - 1) // (NW * CH)) * CH
    nch = ept // CH
    epad = ept * NW

    src = edge_index[0].astype(jnp.int32)
    dst = edge_index[1].astype(jnp.int32)
    src_p = jnp.concatenate(
        [src, jnp.zeros((epad - e,), jnp.int32)]).reshape(NW, ept)
    pad_dst = n + (jnp.arange(epad - e, dtype=jnp.int32) % (npad - n))
    dst_p = jnp.concatenate([dst, pad_dst]).reshape(NW, nch, CH)

    onesf = jnp.ones((CH, feat), jnp.float32)
    zerosf = jnp.zeros((rpt, feat), jnp.float32)

    deg_parts = _deg_kernel_fn(npad, nch, rpt, feat)(dst_p, onesf, zerosf)

    mblk = n // 5
    hs = pl.pallas_call(
        _mm_body,
        grid=(n // mblk,),
        in_specs=[
            pl.BlockSpec((mblk, feat), lambda i: (i, 0)),
            pl.BlockSpec((feat, feat), lambda i: (0, 0)),
            pl.BlockSpec((NC, mblk, feat), lambda i: (0, i, 0)),
        ],
        out_specs=pl.BlockSpec((mblk, feat), lambda i: (i, 0)),
        out_shape=jax.ShapeDtypeStruct((n, feat), jnp.float32),
    )(x, W, deg_parts)

    acc_parts = _scatter_kernel_fn(npad, nch, rpt, ept, feat)(
        hs, src_p, dst_p, zerosf)

    out = pl.pallas_call(
        _combine_body,
        grid=(n // mblk,),
        in_specs=[
            pl.BlockSpec((NC, mblk, feat), lambda i: (0, i, 0)),
            pl.BlockSpec((NC, mblk, feat), lambda i: (0, i, 0)),
            pl.BlockSpec((mblk, feat), lambda i: (i, 0)),
            pl.BlockSpec((1, feat), lambda i: (0, 0)),
        ],
        out_specs=pl.BlockSpec((mblk, feat), lambda i: (i, 0)),
        out_shape=jax.ShapeDtypeStruct((n, feat), jnp.float32),
    )(acc_parts, deg_parts, hs, b.reshape(1, feat))

    return out


# speedup vs baseline: 1.5447x; 1.5447x over previous
import functools

import jax
import jax.numpy as jnp
from jax import lax
from jax.experimental import pallas as pl
from jax.experimental.pallas import tpu as pltpu
from jax.experimental.pallas import tpu_sc as plsc

NC, NS = 2, 16
NW = NC * NS
CH = 104

_MESH = plsc.VectorSubcoreMesh(
    core_axis_name="c", subcore_axis_name="s", num_cores=NC, num_subcores=NS
)


def _deg_kernel_fn(npad, nch, rpt, feat):
    return _deg_kernel_fn2(npad, nch, rpt, feat, jnp.float32)


def _deg_kernel_fn2(npad, nch, rpt, feat, dt):
    @functools.partial(
        pl.kernel,
        out_type=jax.ShapeDtypeStruct((NC, npad, feat), dt),
        mesh=_MESH,
        scratch_types=[
            pltpu.VMEM((nch, CH), jnp.int32),
            pltpu.VMEM((CH, feat), dt),
            pltpu.VMEM_SHARED((npad, feat), dt),
        ],
    )
    def deg_kernel(dst_hbm, ones_hbm, zeros_hbm, deg_out, dst_v, ones_v, deg_sh):
        c = lax.axis_index("c")
        s = lax.axis_index("s")
        gt = c * NS + s
        pltpu.sync_copy(dst_hbm.at[gt], dst_v)
        pltpu.sync_copy(ones_hbm, ones_v)
        base = pl.multiple_of(s * rpt, 8)
        pltpu.sync_copy(zeros_hbm.at[pl.ds(base, rpt)],
                        deg_sh.at[pl.ds(base, rpt)])
        plsc.subcore_barrier()

        def body(j, carry):
            pltpu.sync_copy(ones_v, deg_sh.at[dst_v.at[j]], add=True)
            return carry

        lax.fori_loop(0, nch, body, 0)
        plsc.subcore_barrier()
        pltpu.sync_copy(deg_sh.at[pl.ds(base, rpt)],
                        deg_out.at[c, pl.ds(base, rpt)])

    return deg_kernel


def _scatter_kernel_fn(npad, nch, rpt, ept, feat):
    @functools.partial(
        pl.kernel,
        out_type=jax.ShapeDtypeStruct((NC, npad, feat), jnp.float32),
        mesh=_MESH,
        scratch_types=[
            pltpu.VMEM((ept,), jnp.int32),
            pltpu.VMEM((nch, CH), jnp.int32),
            pltpu.VMEM((2, CH, feat), jnp.float32),
            pltpu.VMEM_SHARED((npad, feat), jnp.float32),
            pltpu.SemaphoreType.DMA,
        ],
    )
    def scatter_kernel(hs_hbm, src_hbm, dst_hbm, zeros_hbm, acc_out,
                       src_v, dst_v, rows_v, acc_sh, sem_g):
        c = lax.axis_index("c")
        s = lax.axis_index("s")
        gt = c * NS + s
        hs_c = hs_hbm.at[c]
        pltpu.sync_copy(src_hbm.at[gt], src_v)
        pltpu.sync_copy(dst_hbm.at[gt], dst_v)
        base = pl.multiple_of(s * rpt, 8)
        pltpu.sync_copy(zeros_hbm.at[pl.ds(base, rpt)],
                        acc_sh.at[pl.ds(base, rpt)])
        plsc.subcore_barrier()

        def issue(j, par):
            off = pl.multiple_of(j * CH, 8)
            pltpu.async_copy(hs_c.at[src_v.at[pl.ds(off, CH)]],
                             rows_v.at[par], sem_g)

        def wait(j, par):
            off = pl.multiple_of(j * CH, 8)
            pltpu.make_async_copy(
                hs_c.at[src_v.at[pl.ds(off, CH)]],
                rows_v.at[par], sem_g).wait()

        issue(0, 0)

        def body(j, carry):
            par = lax.rem(j, 2)
            wait(j, par)
            issue(j + 1, 1 - par)
            pltpu.sync_copy(rows_v.at[par], acc_sh.at[dst_v.at[j]],
                            add=True)
            return carry

        lax.fori_loop(0, nch - 1, body, 0)
        last = nch - 1
        lpar = lax.rem(last, 2)
        wait(last, lpar)
        pltpu.sync_copy(rows_v.at[lpar], acc_sh.at[dst_v.at[last]], add=True)
        plsc.subcore_barrier()
        pltpu.sync_copy(acc_sh.at[pl.ds(base, rpt)],
                        acc_out.at[c, pl.ds(base, rpt)])

    return scatter_kernel


def _scale_body(x_ref, dp_ref, xs_ref):
    dis = lax.rsqrt(dp_ref[0][:, 0:1] + dp_ref[1][:, 0:1] + 1.0)
    v = x_ref[...] * dis
    xs_ref[0] = v
    xs_ref[1] = v


def _combine_body(p_ref, dp_ref, xs_ref, w_ref, b_ref, o_ref):
    dis = lax.rsqrt(dp_ref[0][:, 0:1] + dp_ref[1][:, 0:1] + 1.0)
    agg = dis * (p_ref[0] + p_ref[1] + xs_ref[0])
    o_ref[...] = jnp.dot(agg, w_ref[...],
                         preferred_element_type=jnp.float32) + b_ref[...]


def kernel(x, edge_index, W, b):
    n, feat = x.shape
    e = edge_index.shape[1]
    npad = ((n + 1 + NS * 8 - 1) // (NS * 8)) * (NS * 8)
    rpt = npad // NS
    ept = ((e + NW * CH - 1) // (NW * CH)) * CH
    nch = ept // CH
    epad = ept * NW

    src = edge_index[0].astype(jnp.int32)
    dst = edge_index[1].astype(jnp.int32)
    e2 = ((e + NW - 1) // NW) * NW
    src2 = jnp.concatenate(
        [src, jnp.zeros((e2 - e,), jnp.int32)]).reshape(NW, e2 // NW)
    dst2 = jnp.concatenate(
        [dst, jnp.full((e2 - e,), npad - 1, jnp.int32)]).reshape(NW, e2 // NW)
    ppt = ept - e2 // NW
    pad_rows = jnp.arange(NW * ppt, dtype=jnp.int32).reshape(NW, ppt)
    src_p = jnp.concatenate([src2, pad_rows % n], axis=1)
    dst_p = jnp.concatenate(
        [dst2, n + pad_rows % (npad - n)], axis=1).reshape(NW, nch, CH)

    onesf = jnp.ones((CH, feat), jnp.float32)
    zerosf = jnp.zeros((npad, feat), jnp.float32)

    deg_parts = _deg_kernel_fn(npad, nch, rpt, feat)(dst_p, onesf, zerosf)

    mblk = n // 5
    xs = pl.pallas_call(
        _scale_body,
        grid=(n // mblk,),
        in_specs=[
            pl.BlockSpec((mblk, feat), lambda i: (i, 0)),
            pl.BlockSpec((NC, mblk, feat), lambda i: (0, i, 0)),
        ],
        out_specs=pl.BlockSpec((NC, mblk, feat), lambda i: (0, i, 0)),
        out_shape=jax.ShapeDtypeStruct((NC, n, feat), jnp.float32),
    )(x, deg_parts)

    acc_parts = _scatter_kernel_fn(npad, nch, rpt, ept, feat)(
        xs, src_p, dst_p, zerosf)

    out = pl.pallas_call(
        _combine_body,
        grid=(n // mblk,),
        in_specs=[
            pl.BlockSpec((NC, mblk, feat), lambda i: (0, i, 0)),
            pl.BlockSpec((NC, mblk, feat), lambda i: (0, i, 0)),
            pl.BlockSpec((1, mblk, feat), lambda i: (0, i, 0)),
            pl.BlockSpec((feat, feat), lambda i: (0, 0)),
            pl.BlockSpec((1, feat), lambda i: (0, 0)),
        ],
        out_specs=pl.BlockSpec((mblk, feat), lambda i: (i, 0)),
        out_shape=jax.ShapeDtypeStruct((n, feat), jnp.float32),
    )(acc_parts, deg_parts, xs, W, b.reshape(1, feat))

    return out


# speedup vs baseline: 1.5554x; 1.0069x over previous
import functools

import jax
import jax.numpy as jnp
from jax import lax
from jax.experimental import pallas as pl
from jax.experimental.pallas import tpu as pltpu
from jax.experimental.pallas import tpu_sc as plsc

NC, NS = 2, 16
NW = NC * NS
CH = 104

_MESH = plsc.VectorSubcoreMesh(
    core_axis_name="c", subcore_axis_name="s", num_cores=NC, num_subcores=NS
)


def _deg_kernel_fn(npad, nch, rpt, feat):
    return _deg_kernel_fn2(npad, nch, rpt, feat, jnp.float32)


def _deg_kernel_fn2(npad, nch, rpt, feat, dt):
    @functools.partial(
        pl.kernel,
        out_type=jax.ShapeDtypeStruct((NC, npad, feat), dt),
        mesh=_MESH,
        scratch_types=[
            pltpu.VMEM((nch, CH), jnp.int32),
            pltpu.VMEM((CH, feat), dt),
            pltpu.VMEM_SHARED((npad, feat), dt),
        ],
    )
    def deg_kernel(dst_hbm, ones_hbm, zeros_hbm, deg_out, dst_v, ones_v, deg_sh):
        c = lax.axis_index("c")
        s = lax.axis_index("s")
        gt = c * NS + s
        pltpu.sync_copy(dst_hbm.at[gt], dst_v)
        pltpu.sync_copy(ones_hbm, ones_v)
        base = pl.multiple_of(s * rpt, 8)
        pltpu.sync_copy(zeros_hbm.at[pl.ds(base, rpt)],
                        deg_sh.at[pl.ds(base, rpt)])
        plsc.subcore_barrier()

        def body(j, carry):
            pltpu.sync_copy(ones_v, deg_sh.at[dst_v.at[j]], add=True)
            return carry

        lax.fori_loop(0, nch, body, 0)
        plsc.subcore_barrier()
        pltpu.sync_copy(deg_sh.at[pl.ds(base, rpt)],
                        deg_out.at[c, pl.ds(base, rpt)])

    return deg_kernel


def _scatter_kernel_fn(npad, nch, rpt, ept, feat):
    @functools.partial(
        pl.kernel,
        out_type=jax.ShapeDtypeStruct((NC, npad, feat), jnp.float32),
        mesh=_MESH,
        scratch_types=[
            pltpu.VMEM((ept,), jnp.int32),
            pltpu.VMEM((nch, CH), jnp.int32),
            pltpu.VMEM((2, CH, feat), jnp.float32),
            pltpu.VMEM_SHARED((npad, feat), jnp.float32),
            pltpu.SemaphoreType.DMA,
        ],
    )
    def scatter_kernel(hs_hbm, src_hbm, dst_hbm, zeros_hbm, acc_out,
                       src_v, dst_v, rows_v, acc_sh, sem_g):
        c = lax.axis_index("c")
        s = lax.axis_index("s")
        gt = c * NS + s
        pltpu.sync_copy(src_hbm.at[gt], src_v)
        pltpu.sync_copy(dst_hbm.at[gt], dst_v)
        base = pl.multiple_of(s * rpt, 8)
        pltpu.sync_copy(zeros_hbm.at[pl.ds(base, rpt)],
                        acc_sh.at[pl.ds(base, rpt)])
        plsc.subcore_barrier()

        def issue(j, par):
            off = pl.multiple_of(j * CH, 8)
            pltpu.async_copy(hs_hbm.at[src_v.at[pl.ds(off, CH)]],
                             rows_v.at[par], sem_g)

        def wait(j, par):
            off = pl.multiple_of(j * CH, 8)
            pltpu.make_async_copy(
                hs_hbm.at[src_v.at[pl.ds(off, CH)]],
                rows_v.at[par], sem_g).wait()

        issue(0, 0)

        def body(j, carry):
            par = lax.rem(j, 2)
            wait(j, par)
            issue(j + 1, 1 - par)
            pltpu.sync_copy(rows_v.at[par], acc_sh.at[dst_v.at[j]],
                            add=True)
            return carry

        lax.fori_loop(0, nch - 1, body, 0)
        last = nch - 1
        lpar = lax.rem(last, 2)
        wait(last, lpar)
        pltpu.sync_copy(rows_v.at[lpar], acc_sh.at[dst_v.at[last]], add=True)
        plsc.subcore_barrier()
        pltpu.sync_copy(acc_sh.at[pl.ds(base, rpt)],
                        acc_out.at[c, pl.ds(base, rpt)])

    return scatter_kernel


def _scale_body(x_ref, dp_ref, xs_ref):
    dis = lax.rsqrt(dp_ref[0][:, 0:1] + dp_ref[1][:, 0:1] + 1.0)
    xs_ref[...] = x_ref[...] * dis


def _combine_body(p_ref, dp_ref, xs_ref, w_ref, b_ref, o_ref):
    dis = lax.rsqrt(dp_ref[0][:, 0:1] + dp_ref[1][:, 0:1] + 1.0)
    agg = dis * (p_ref[0] + p_ref[1] + xs_ref[...])
    o_ref[...] = jnp.dot(agg, w_ref[...],
                         preferred_element_type=jnp.float32) + b_ref[...]


def kernel(x, edge_index, W, b):
    n, feat = x.shape
    e = edge_index.shape[1]
    npad = ((n + 1 + NS * 8 - 1) // (NS * 8)) * (NS * 8)
    rpt = npad // NS
    ept = ((e + NW * CH - 1) // (NW * CH)) * CH
    nch = ept // CH
    epad = ept * NW

    src = edge_index[0].astype(jnp.int32)
    dst = edge_index[1].astype(jnp.int32)
    e2 = ((e + NW - 1) // NW) * NW
    src2 = jnp.concatenate(
        [src, jnp.zeros((e2 - e,), jnp.int32)]).reshape(NW, e2 // NW)
    dst2 = jnp.concatenate(
        [dst, jnp.full((e2 - e,), npad - 1, jnp.int32)]).reshape(NW, e2 // NW)
    ppt = ept - e2 // NW
    pad_rows = jnp.arange(NW * ppt, dtype=jnp.int32).reshape(NW, ppt)
    src_p = jnp.concatenate([src2, pad_rows % n], axis=1)
    dst_p = jnp.concatenate(
        [dst2, n + pad_rows % (npad - n)], axis=1).reshape(NW, nch, CH)

    onesf = jnp.ones((CH, feat), jnp.float32)
    zerosf = jnp.zeros((npad, feat), jnp.float32)

    deg_parts = _deg_kernel_fn(npad, nch, rpt, feat)(dst_p, onesf, zerosf)

    mblk = n // 5
    xs = pl.pallas_call(
        _scale_body,
        grid=(n // mblk,),
        in_specs=[
            pl.BlockSpec((mblk, feat), lambda i: (i, 0)),
            pl.BlockSpec((NC, mblk, feat), lambda i: (0, i, 0)),
        ],
        out_specs=pl.BlockSpec((mblk, feat), lambda i: (i, 0)),
        out_shape=jax.ShapeDtypeStruct((n, feat), jnp.float32),
    )(x, deg_parts)

    acc_parts = _scatter_kernel_fn(npad, nch, rpt, ept, feat)(
        xs, src_p, dst_p, zerosf)

    out = pl.pallas_call(
        _combine_body,
        grid=(n // mblk,),
        in_specs=[
            pl.BlockSpec((NC, mblk, feat), lambda i: (0, i, 0)),
            pl.BlockSpec((NC, mblk, feat), lambda i: (0, i, 0)),
            pl.BlockSpec((mblk, feat), lambda i: (i, 0)),
            pl.BlockSpec((feat, feat), lambda i: (0, 0)),
            pl.BlockSpec((1, feat), lambda i: (0, 0)),
        ],
        out_specs=pl.BlockSpec((mblk, feat), lambda i: (i, 0)),
        out_shape=jax.ShapeDtypeStruct((n, feat), jnp.float32),
    )(acc_parts, deg_parts, xs, W, b.reshape(1, feat))

    return out
